# Initial kernel scaffold; baseline (speedup 1.0000x reference)
import jax
import jax.numpy as jnp
from jax.experimental import pallas as pl


def kernel(dataset_ids, image_ids, patch_ids, W_dataset, W_image, W_patch, W_proj, b_proj):
    raise NotImplementedError("write your pallas kernel here")


# speedup vs baseline: 2.1741x; 2.1741x over previous
import functools

import jax
import jax.numpy as jnp
from jax import lax
from jax.experimental import pallas as pl
from jax.experimental.pallas import tpu as pltpu
from jax.experimental.pallas import tpu_sc as plsc

EMBED_DIM = 64
MODEL_DIM = 128
_CH = 128


def _sc_gather(did, iid, pid, wd, wi, wp):
    info = plsc.get_sparse_core_info()
    nc, ns = info.num_cores, info.num_subcores
    nw = nc * ns
    tok = did.shape[0]
    per_w = tok // nw
    nch = per_w // _CH

    @functools.partial(
        pl.kernel,
        mesh=plsc.VectorSubcoreMesh(core_axis_name="c", subcore_axis_name="s"),
        compiler_params=pltpu.CompilerParams(use_tc_tiling_on_sc=False),
        out_type=[
            jax.ShapeDtypeStruct((tok, EMBED_DIM), jnp.float32),
            jax.ShapeDtypeStruct((tok, EMBED_DIM), jnp.float32),
            jax.ShapeDtypeStruct((tok, EMBED_DIM), jnp.float32),
        ],
        scratch_types=[
            pltpu.VMEM((per_w,), jnp.int32),
            pltpu.VMEM((per_w,), jnp.int32),
            pltpu.VMEM((per_w,), jnp.int32),
            pltpu.VMEM((_CH, EMBED_DIM), jnp.float32),
            pltpu.VMEM((_CH, EMBED_DIM), jnp.float32),
            pltpu.VMEM((_CH, EMBED_DIM), jnp.float32),
            pltpu.SemaphoreType.DMA,
        ],
    )
    def k(did_h, iid_h, pid_h, wd_h, wi_h, wp_h, dg_h, ig_h, pg_h,
          xd, xi, xp, bd, bi, bp, sem):
        wid = lax.axis_index("s") * nc + lax.axis_index("c")
        base = wid * per_w
        pltpu.sync_copy(did_h.at[pl.ds(base, per_w)], xd)
        pltpu.sync_copy(iid_h.at[pl.ds(base, per_w)], xi)
        pltpu.sync_copy(pid_h.at[pl.ds(base, per_w)], xp)

        def body(ch, carry):
            off = ch * _CH
            cd = pltpu.async_copy(wd_h.at[xd.at[pl.ds(off, _CH)]], bd, sem)
            ci = pltpu.async_copy(wi_h.at[xi.at[pl.ds(off, _CH)]], bi, sem)
            cp = pltpu.async_copy(wp_h.at[xp.at[pl.ds(off, _CH)]], bp, sem)
            cd.wait()
            ci.wait()
            cp.wait()
            g = base + off
            pltpu.sync_copy(bd, dg_h.at[pl.ds(g, _CH)])
            pltpu.sync_copy(bi, ig_h.at[pl.ds(g, _CH)])
            pltpu.sync_copy(bp, pg_h.at[pl.ds(g, _CH)])
            return carry

        lax.fori_loop(0, nch, body, 0)

    return k(did, iid, pid, wd, wi, wp)


def _tc_proj(dg, ig, pg, w_proj, b_proj):
    tok = dg.shape[0]
    br = 1024
    e = EMBED_DIM

    def body(d_ref, i_ref, p_ref, w_ref, b_ref, o_ref):
        acc = jnp.dot(d_ref[...], w_ref[0:e, :], preferred_element_type=jnp.float32)
        acc += jnp.dot(i_ref[...], w_ref[e:2 * e, :], preferred_element_type=jnp.float32)
        acc += jnp.dot(p_ref[...], w_ref[2 * e:3 * e, :], preferred_element_type=jnp.float32)
        o_ref[...] = acc + b_ref[...]

    return pl.pallas_call(
        body,
        grid=(tok // br,),
        in_specs=[
            pl.BlockSpec((br, e), lambda i: (i, 0)),
            pl.BlockSpec((br, e), lambda i: (i, 0)),
            pl.BlockSpec((br, e), lambda i: (i, 0)),
            pl.BlockSpec((3 * e, MODEL_DIM), lambda i: (0, 0)),
            pl.BlockSpec((1, MODEL_DIM), lambda i: (0, 0)),
        ],
        out_specs=pl.BlockSpec((br, MODEL_DIM), lambda i: (i, 0)),
        out_shape=jax.ShapeDtypeStruct((tok, MODEL_DIM), jnp.float32),
    )(dg, ig, pg, w_proj, b_proj.reshape(1, MODEL_DIM))


def kernel(dataset_ids, image_ids, patch_ids, W_dataset, W_image, W_patch,
           W_proj, b_proj):
    b, l = dataset_ids.shape
    did = dataset_ids.reshape(-1).astype(jnp.int32)
    iid = image_ids.reshape(-1).astype(jnp.int32)
    pid = patch_ids.reshape(-1).astype(jnp.int32)
    dg, ig, pg = _sc_gather(did, iid, pid, W_dataset, W_image, W_patch)
    out = _tc_proj(dg, ig, pg, W_proj, b_proj)
    return out.reshape(b, l, MODEL_DIM)


# speedup vs baseline: 3.4765x; 1.5991x over previous
import functools

import jax
import jax.numpy as jnp
from jax import lax
from jax.experimental import pallas as pl
from jax.experimental.pallas import tpu as pltpu
from jax.experimental.pallas import tpu_sc as plsc

EMBED_DIM = 64
MODEL_DIM = 128
_CH = 128
_LANES = 16


def _tc_preproject(tbl, w, bias=None, block_rows=None):
    n = tbl.shape[0]
    br = block_rows or n

    if bias is None:
        def body(t_ref, w_ref, o_ref):
            o_ref[...] = jnp.dot(t_ref[...], w_ref[...],
                                 preferred_element_type=jnp.float32)
        extra_in, extra_spec = (), ()
    else:
        def body(t_ref, w_ref, b_ref, o_ref):
            o_ref[...] = jnp.dot(t_ref[...], w_ref[...],
                                 preferred_element_type=jnp.float32) + b_ref[...]
        extra_in = (bias.reshape(1, MODEL_DIM),)
        extra_spec = (pl.BlockSpec((1, MODEL_DIM), lambda i: (0, 0)),)

    return pl.pallas_call(
        body,
        grid=(n // br,),
        in_specs=[
            pl.BlockSpec((br, EMBED_DIM), lambda i: (i, 0)),
            pl.BlockSpec((EMBED_DIM, MODEL_DIM), lambda i: (0, 0)),
            *extra_spec,
        ],
        out_specs=pl.BlockSpec((br, MODEL_DIM), lambda i: (i, 0)),
        out_shape=jax.ShapeDtypeStruct((n, MODEL_DIM), jnp.float32),
    )(tbl, w, *extra_in)


def _sc_gather_sum(did, iid, pid, pd, pp, pi):
    info = plsc.get_sparse_core_info()
    nc, ns = info.num_cores, info.num_subcores
    nw = nc * ns
    tok = did.shape[0]
    per_w = tok // nw
    idb = 2560
    nblk = per_w // idb
    nch = idb // _CH
    nd, np_ = pd.shape[0], pp.shape[0]

    @functools.partial(
        pl.kernel,
        mesh=plsc.VectorSubcoreMesh(core_axis_name="c", subcore_axis_name="s"),
        out_type=jax.ShapeDtypeStruct((tok, MODEL_DIM), jnp.float32),
        scratch_types=[
            pltpu.VMEM((idb,), jnp.int32),
            pltpu.VMEM((idb,), jnp.int32),
            pltpu.VMEM((idb,), jnp.int32),
            pltpu.VMEM((_CH, MODEL_DIM), jnp.float32),
            pltpu.VMEM((_CH, MODEL_DIM), jnp.float32),
            pltpu.VMEM((_CH, MODEL_DIM), jnp.float32),
            pltpu.SemaphoreType.DMA,
        ],
    )
    def k(did_h, iid_h, pid_h, pd_h, pp_h, pi_h, out_h,
          xd, xi, xp, gd, gi, gp, sem):
        sid = lax.axis_index("s")
        wid = sid * nc + lax.axis_index("c")
        base = wid * per_w

        def blk(bi_, carry):
            boff = base + bi_ * idb
            pltpu.sync_copy(did_h.at[pl.ds(boff, idb)], xd)
            pltpu.sync_copy(iid_h.at[pl.ds(boff, idb)], xi)
            pltpu.sync_copy(pid_h.at[pl.ds(boff, idb)], xp)

            def chunk(ch, c1):
                off = ch * _CH
                ci = pltpu.async_copy(pi_h.at[xi.at[pl.ds(off, _CH)]], gi, sem)
                cd = pltpu.async_copy(pd_h.at[xd.at[pl.ds(off, _CH)]], gd, sem)
                cp = pltpu.async_copy(pp_h.at[xp.at[pl.ds(off, _CH)]], gp, sem)
                ci.wait()
                cd.wait()
                cp.wait()

                def row(j, c2):
                    for kk in range(MODEL_DIM // _LANES):
                        s = pl.ds(kk * _LANES, _LANES)
                        plsc.addupdate(gi.at[j, s], gd[j, s])
                        plsc.addupdate(gi.at[j, s], gp[j, s])
                    return c2

                lax.fori_loop(0, _CH, row, 0)
                pltpu.sync_copy(gi, out_h.at[pl.ds(boff + off, _CH)])
                return c1

            lax.fori_loop(0, nch, chunk, 0)
            return carry

        lax.fori_loop(0, nblk, blk, 0)

    return k(did, iid, pid, pd, pp, pi)


def kernel(dataset_ids, image_ids, patch_ids, W_dataset, W_image, W_patch,
           W_proj, b_proj):
    b, l = dataset_ids.shape
    did = dataset_ids.reshape(-1).astype(jnp.int32)
    iid = image_ids.reshape(-1).astype(jnp.int32)
    pid = patch_ids.reshape(-1).astype(jnp.int32)
    e = EMBED_DIM
    pd = _tc_preproject(W_dataset, W_proj[0:e], bias=b_proj)
    pi = _tc_preproject(W_image, W_proj[e:2 * e], block_rows=8000)
    pp = _tc_preproject(W_patch, W_proj[2 * e:3 * e])
    out = _sc_gather_sum(did, iid, pid, pd, pp, pi)
    return out.reshape(b, l, MODEL_DIM)
